# baseline (speedup 1.0000x reference)
import functools

import jax
import jax.numpy as jnp
from jax import lax
from jax.experimental import pallas as pl
from jax.experimental.pallas import tpu as pltpu
from jax.experimental.pallas import tpu_sc as plsc

_LANES = 16
_FRAC0 = 0.93


@functools.partial(jax.jit, static_argnums=(2, 3, 4))
def _gather_mean(idx_flat, table, nb0, nb1, s):
    info = plsc.get_sparse_core_info()
    nc, ns = info.num_cores, info.num_subcores
    d = table.shape[1] * 2
    bp = ns * (nb0 + nb1)
    rpc = 128 // s
    nch0, nch1 = nb0 // rpc, nb1 // rpc

    mesh = plsc.VectorSubcoreMesh(core_axis_name="c", subcore_axis_name="s")

    @functools.partial(
        pl.kernel,
        mesh=mesh,
        out_type=jax.ShapeDtypeStruct((bp, d), jnp.float32),
        compiler_params=pltpu.CompilerParams(needs_layout_passes=False,
                                             use_tc_tiling_on_sc=False),
        scratch_types=[
            pltpu.VMEM((nch0 * 128,), jnp.int32),
            pltpu.VMEM((rpc * s, d // 2), jnp.int32),
            pltpu.VMEM((rpc * s, d // 2), jnp.int32),
            pltpu.VMEM((nb0, d), jnp.float32),
            pltpu.SemaphoreType.DMA,
            pltpu.SemaphoreType.DMA,
        ],
    )
    def k(idx_hbm, table_hbm, out_hbm, idx_v, buf0, buf1, out_v, sem0, sem1):
        cid = lax.axis_index("c")
        sid = lax.axis_index("s")
        is0 = cid == 0
        sid32 = sid.astype(jnp.int32)
        base = jnp.where(is0, sid32 * jnp.int32(nb0),
                         jnp.int32(ns * nb0) + sid32 * jnp.int32(nb1))
        nch = jnp.where(is0, jnp.int32(nch0), jnp.int32(nch1))

        pltpu.sync_copy(idx_hbm.at[pl.ds(base * s, nb1 * s)],
                        idx_v.at[pl.ds(0, nb1 * s)])

        @pl.when(is0)
        def _():
            pltpu.sync_copy(
                idx_hbm.at[pl.ds(base * s + nb1 * s, (nb0 - nb1) * s)],
                idx_v.at[pl.ds(nb1 * s, (nb0 - nb1) * s)])

        scale = jnp.float32(1.0 / s)
        nvec = d // _LANES
        bufs = (buf0, buf1)
        sems = (sem0, sem1)

        def idx_at(c):
            return idx_v.at[pl.ds(c * 128, 128)]

        ngrp = d // 32

        def load_row(j):
            out = []
            for g in range(ngrp):
                w = buf_cur[0][j, pl.ds(g * 16, 16)]
                ab = plsc.bitcast(w, jnp.bfloat16)
                a, b2 = plsc.unpack(ab, format=plsc.PackFormat.INTERLEAVED)
                out.append(a)
                out.append(b2)
            return tuple(out)

        buf_cur = [None]

        def compute(c, buf):
            buf_cur[0] = buf
            for r in range(rpc):
                rbase = jnp.int32(r * s)

                def nb_body(j, accs):
                    vals = load_row(rbase + j)
                    return tuple(a + v for a, v in zip(accs, vals))

                accs = lax.fori_loop(jnp.int32(1), jnp.int32(s), nb_body,
                                     load_row(rbase))
                row = c * rpc + r
                for g in range(ngrp):
                    out_v[row, pl.ds(g * 32, _LANES)] = accs[2 * g] * scale
                    out_v[row, pl.ds(g * 32 + _LANES, _LANES)] = (
                        accs[2 * g + 1] * scale)

        pltpu.async_copy(table_hbm.at[idx_at(jnp.int32(0))], buf0, sem0)

        def body(t, carry):
            for b in range(2):
                c = t * 2 + b
                pltpu.make_async_copy(table_hbm.at[idx_at(c)],
                                      bufs[b], sems[b]).wait()
                nxt = c + 1

                @pl.when(nxt < nch)
                def _():
                    pltpu.async_copy(table_hbm.at[idx_at(nxt)],
                                     bufs[1 - b], sems[1 - b])

                compute(c, bufs[b])
            return carry

        lax.fori_loop(jnp.int32(0), nch // 2, body, jnp.int32(0))

        pltpu.sync_copy(out_v.at[pl.ds(0, nb1)],
                        out_hbm.at[pl.ds(base, nb1)])

        @pl.when(is0)
        def _():
            pltpu.sync_copy(out_v.at[pl.ds(nb1, nb0 - nb1)],
                            out_hbm.at[pl.ds(base + nb1, nb0 - nb1)])

    return k(idx_flat, table)


def kernel(nodes, neighbors, num_sample, emb_weight):
    b, s = neighbors.shape
    idx = neighbors.astype(jnp.int32)

    info = plsc.get_sparse_core_info()
    ns = info.num_subcores
    rpc = max(1, 128 // s)
    grain = max(8, rpc * 2)
    align = ns * grain * 2
    bp = ((b + align - 1) // align) * align
    if bp != b:
        idx = jnp.pad(idx, ((0, bp - b), (0, 0)))
    per_pair = bp // ns
    nb0 = int(round(_FRAC0 * per_pair / grain)) * grain
    nb0 = min(max(nb0, grain), per_pair - grain)
    nb1 = per_pair - nb0

    emb16 = emb_weight.astype(jnp.bfloat16)
    n, d = emb16.shape
    tbl = lax.bitcast_convert_type(emb16.reshape(n, d // 2, 2), jnp.int32)

    out = _gather_mean(idx.reshape(bp * s), tbl, nb0, nb1, s)
    out = out.reshape(bp, d // 32, 2, 16).swapaxes(2, 3).reshape(bp, d)
    return out[:b]


# speedup vs baseline: 1.8529x; 1.8529x over previous
import functools

import jax
import jax.numpy as jnp
from jax import lax
from jax.experimental import pallas as pl
from jax.experimental.pallas import tpu as pltpu
from jax.experimental.pallas import tpu_sc as plsc

_LANES = 16
_FRAC0 = 0.5


@functools.partial(jax.jit, static_argnums=(2, 3, 4))
def _gather_mean(idx_flat, table, nb0, nb1, s):
    info = plsc.get_sparse_core_info()
    nc, ns = info.num_cores, info.num_subcores
    d = table.shape[1]
    bp = ns * (nb0 + nb1)
    rpc = 256 // s
    nch0, nch1 = nb0 // rpc, nb1 // rpc

    mesh = plsc.VectorSubcoreMesh(core_axis_name="c", subcore_axis_name="s")

    @functools.partial(
        pl.kernel,
        mesh=mesh,
        out_type=jax.ShapeDtypeStruct((bp, d), jnp.float32),
        scratch_types=[
            pltpu.VMEM((nch0 * 256,), jnp.int32),
            pltpu.VMEM((rpc * s, d), jnp.float32),
            pltpu.VMEM((rpc * s, d), jnp.float32),
            pltpu.VMEM((nb0, d), jnp.float32),
            pltpu.SemaphoreType.DMA,
            pltpu.SemaphoreType.DMA,
        ],
    )
    def k(idx_hbm, table_hbm, out_hbm, idx_v, buf0, buf1, out_v, sem0, sem1):
        cid = lax.axis_index("c")
        sid = lax.axis_index("s")
        is0 = cid == 0
        sid32 = sid.astype(jnp.int32)
        base = jnp.where(is0, sid32 * jnp.int32(nb0),
                         jnp.int32(ns * nb0) + sid32 * jnp.int32(nb1))
        nch = jnp.where(is0, jnp.int32(nch0), jnp.int32(nch1))

        pltpu.sync_copy(idx_hbm.at[pl.ds(base * s, nb1 * s)],
                        idx_v.at[pl.ds(0, nb1 * s)])

        if nb0 > nb1:
            @pl.when(is0)
            def _():
                pltpu.sync_copy(
                    idx_hbm.at[pl.ds(base * s + nb1 * s, (nb0 - nb1) * s)],
                    idx_v.at[pl.ds(nb1 * s, (nb0 - nb1) * s)])

        scale = jnp.float32(1.0 / s)
        nvec = d // _LANES
        bufs = (buf0, buf1)
        sems = (sem0, sem1)

        def idx_at(c):
            return idx_v.at[pl.ds(c * 256, 256)]

        def compute(c, buf):
            for r in range(rpc):
                rbase = jnp.int32(r * s)

                def nb_body(j, accs):
                    return tuple(
                        accs[v] + buf[rbase + j, pl.ds(v * _LANES, _LANES)]
                        for v in range(nvec))

                accs = tuple(buf[rbase, pl.ds(v * _LANES, _LANES)]
                             for v in range(nvec))
                accs = lax.fori_loop(jnp.int32(1), jnp.int32(s), nb_body,
                                     accs)
                row = c * rpc + r
                for v in range(nvec):
                    out_v[row, pl.ds(v * _LANES, _LANES)] = accs[v] * scale

        pltpu.async_copy(table_hbm.at[idx_at(jnp.int32(0))], buf0, sem0)

        def body(t, carry):
            for b in range(2):
                c = t * 2 + b
                pltpu.make_async_copy(table_hbm.at[idx_at(c)],
                                      bufs[b], sems[b]).wait()
                nxt = c + 1

                @pl.when(nxt < nch)
                def _():
                    pltpu.async_copy(table_hbm.at[idx_at(nxt)],
                                     bufs[1 - b], sems[1 - b])

                compute(c, bufs[b])
            return carry

        lax.fori_loop(jnp.int32(0), nch // 2, body, jnp.int32(0))

        pltpu.sync_copy(out_v.at[pl.ds(0, nb1)],
                        out_hbm.at[pl.ds(base, nb1)])

        if nb0 > nb1:
            @pl.when(is0)
            def _():
                pltpu.sync_copy(out_v.at[pl.ds(nb1, nb0 - nb1)],
                                out_hbm.at[pl.ds(base + nb1, nb0 - nb1)])

    return k(idx_flat, table)


def kernel(nodes, neighbors, num_sample, emb_weight):
    b, s = neighbors.shape
    idx = neighbors.astype(jnp.int32)

    info = plsc.get_sparse_core_info()
    ns = info.num_subcores
    rpc = max(1, 256 // s)
    grain = max(8, rpc * 2)
    align = ns * grain * 2
    bp = ((b + align - 1) // align) * align
    if bp != b:
        idx = jnp.pad(idx, ((0, bp - b), (0, 0)))
    per_pair = bp // ns
    nb0 = int(round(_FRAC0 * per_pair / grain)) * grain
    nb0 = min(max(nb0, grain), per_pair - grain)
    nb1 = per_pair - nb0

    out = _gather_mean(idx.reshape(bp * s), emb_weight.astype(jnp.float32),
                       nb0, nb1, s)
    return out[:b]


# speedup vs baseline: 1.8727x; 1.0106x over previous
import functools

import jax
import jax.numpy as jnp
from jax import lax
from jax.experimental import pallas as pl
from jax.experimental.pallas import tpu as pltpu
from jax.experimental.pallas import tpu_sc as plsc

_LANES = 16
_FRAC0 = 0.63


@functools.partial(jax.jit, static_argnums=(2, 3, 4))
def _gather_mean(idx_flat, table, nb0, nb1, s):
    info = plsc.get_sparse_core_info()
    nc, ns = info.num_cores, info.num_subcores
    d = table.shape[1]
    bp = ns * (nb0 + nb1)
    rpc = 256 // s
    nch0, nch1 = nb0 // rpc, nb1 // rpc

    mesh = plsc.VectorSubcoreMesh(core_axis_name="c", subcore_axis_name="s")

    @functools.partial(
        pl.kernel,
        mesh=mesh,
        out_type=jax.ShapeDtypeStruct((bp, d), jnp.float32),
        scratch_types=[
            pltpu.VMEM((nch0 * 256,), jnp.int32),
            pltpu.VMEM((rpc * s, d), jnp.float32),
            pltpu.VMEM((rpc * s, d), jnp.float32),
            pltpu.VMEM((nb0, d), jnp.float32),
            pltpu.SemaphoreType.DMA,
            pltpu.SemaphoreType.DMA,
        ],
    )
    def k(idx_hbm, table_hbm, out_hbm, idx_v, buf0, buf1, out_v, sem0, sem1):
        cid = lax.axis_index("c")
        sid = lax.axis_index("s")
        is0 = cid == 0
        sid32 = sid.astype(jnp.int32)
        base = jnp.where(is0, sid32 * jnp.int32(nb0),
                         jnp.int32(ns * nb0) + sid32 * jnp.int32(nb1))
        nch = jnp.where(is0, jnp.int32(nch0), jnp.int32(nch1))

        pltpu.sync_copy(idx_hbm.at[pl.ds(base * s, nb1 * s)],
                        idx_v.at[pl.ds(0, nb1 * s)])

        if nb0 > nb1:
            @pl.when(is0)
            def _():
                pltpu.sync_copy(
                    idx_hbm.at[pl.ds(base * s + nb1 * s, (nb0 - nb1) * s)],
                    idx_v.at[pl.ds(nb1 * s, (nb0 - nb1) * s)])

        scale = jnp.float32(1.0 / s)
        nvec = d // _LANES
        bufs = (buf0, buf1)
        sems = (sem0, sem1)

        def idx_at(c):
            return idx_v.at[pl.ds(c * 256, 256)]

        def compute(c, buf):
            for r in range(rpc):
                rbase = jnp.int32(r * s)

                def nb_body(j, accs):
                    return tuple(
                        accs[v] + buf[rbase + j, pl.ds(v * _LANES, _LANES)]
                        for v in range(nvec))

                accs = tuple(buf[rbase, pl.ds(v * _LANES, _LANES)]
                             for v in range(nvec))
                accs = lax.fori_loop(jnp.int32(1), jnp.int32(s), nb_body,
                                     accs)
                row = c * rpc + r
                for v in range(nvec):
                    out_v[row, pl.ds(v * _LANES, _LANES)] = accs[v] * scale

        pltpu.async_copy(table_hbm.at[idx_at(jnp.int32(0))], buf0, sem0)

        def body(t, carry):
            for b in range(2):
                c = t * 2 + b
                pltpu.make_async_copy(table_hbm.at[idx_at(c)],
                                      bufs[b], sems[b]).wait()
                nxt = c + 1

                @pl.when(nxt < nch)
                def _():
                    pltpu.async_copy(table_hbm.at[idx_at(nxt)],
                                     bufs[1 - b], sems[1 - b])

                compute(c, bufs[b])
            return carry

        lax.fori_loop(jnp.int32(0), nch // 2, body, jnp.int32(0))

        pltpu.sync_copy(out_v.at[pl.ds(0, nb1)],
                        out_hbm.at[pl.ds(base, nb1)])

        if nb0 > nb1:
            @pl.when(is0)
            def _():
                pltpu.sync_copy(out_v.at[pl.ds(nb1, nb0 - nb1)],
                                out_hbm.at[pl.ds(base + nb1, nb0 - nb1)])

    return k(idx_flat, table)


def kernel(nodes, neighbors, num_sample, emb_weight):
    b, s = neighbors.shape
    idx = neighbors.astype(jnp.int32)

    info = plsc.get_sparse_core_info()
    ns = info.num_subcores
    rpc = max(1, 256 // s)
    grain = max(8, rpc * 2)
    align = ns * grain * 2
    bp = ((b + align - 1) // align) * align
    if bp != b:
        idx = jnp.pad(idx, ((0, bp - b), (0, 0)))
    per_pair = bp // ns
    nb0 = int(round(_FRAC0 * per_pair / grain)) * grain
    nb0 = min(max(nb0, grain), per_pair - grain)
    nb1 = per_pair - nb0

    out = _gather_mean(idx.reshape(bp * s), emb_weight.astype(jnp.float32),
                       nb0, nb1, s)
    return out[:b]
